# baseline (speedup 1.0000x reference)
import functools

import jax
import jax.numpy as jnp
from jax import lax
from jax.experimental import pallas as pl
from jax.experimental.pallas import tpu as pltpu
from jax.experimental.pallas import tpu_sc as plsc

_FULL_ROWS = 100_000
_FULL_COLS = 128
_N_ALLOC = _FULL_ROWS * _FULL_COLS
_NUM_BANK = 1_280_000
_NUM_CORES = 2
_NUM_SUBCORES = 16
_NW = _NUM_CORES * _NUM_SUBCORES
_PER_W = _N_ALLOC // _NW
_CHUNK = 2_000
_NCHUNK = _PER_W // _CHUNK
_NBUF = 8
_GLAG = _NBUF // 2
_NITER = _NCHUNK // _NBUF
_BANK_SLICE = _NUM_BANK // _NUM_SUBCORES


def _make_gather():
    mesh = plsc.VectorSubcoreMesh(
        core_axis_name="c", subcore_axis_name="s")

    @functools.partial(
        pl.kernel,
        mesh=mesh,
        out_type=jax.ShapeDtypeStruct((_N_ALLOC,), jnp.float32),
        scratch_types=(
            [pltpu.VMEM_SHARED((_NUM_BANK,), jnp.float32)]
            + [pltpu.VMEM((_CHUNK,), jnp.int32) for _ in range(_NBUF)]
            + [pltpu.VMEM((_CHUNK,), jnp.float32) for _ in range(_NBUF)]
            + [pltpu.SemaphoreType.DMA for _ in range(3 * _NBUF)]
        ),
    )
    def gather_kernel(bank_hbm, wa_hbm, out_hbm, bank_sh, *bufs):
        idx_v = bufs[0:_NBUF]
        rows_v = bufs[_NBUF:2 * _NBUF]
        sem_i = bufs[2 * _NBUF:3 * _NBUF]
        sem_o = bufs[3 * _NBUF:4 * _NBUF]
        sem_g = bufs[4 * _NBUF:5 * _NBUF]

        sid = lax.axis_index("s")
        wid = lax.axis_index("c") * _NUM_SUBCORES + sid
        base = wid * _PER_W

        def idx_start(k, j):
            pltpu.async_copy(
                wa_hbm.at[pl.ds(base + k * _CHUNK, _CHUNK)],
                idx_v[j], sem_i[j])

        def idx_wait(j):
            pltpu.make_async_copy(
                wa_hbm.at[pl.ds(base, _CHUNK)], idx_v[j], sem_i[j]).wait()

        def gather_start(j):
            pltpu.async_copy(bank_sh.at[idx_v[j]], rows_v[j], sem_g[j])

        def gather_wait(j):
            pltpu.make_async_copy(
                bank_sh.at[idx_v[j]], rows_v[j], sem_g[j]).wait()

        def out_start(k, j):
            pltpu.async_copy(
                rows_v[j], out_hbm.at[pl.ds(base + k * _CHUNK, _CHUNK)],
                sem_o[j])

        def out_wait(j):
            pltpu.make_async_copy(
                rows_v[j], out_hbm.at[pl.ds(base, _CHUNK)], sem_o[j]).wait()

        for j in range(_GLAG):
            idx_start(j, j)

        boff = sid * _BANK_SLICE
        pltpu.sync_copy(bank_hbm.at[pl.ds(boff, _BANK_SLICE)],
                        bank_sh.at[pl.ds(boff, _BANK_SLICE)])
        plsc.subcore_barrier()

        def body(i, carry):
            for j in range(_NBUF):
                k = i * _NBUF + j
                jl = (j + _GLAG) % _NBUF

                idx_wait(j)

                @pl.when(k >= _NBUF)
                def _():
                    out_wait(j)

                gather_start(j)

                @pl.when(k >= _GLAG)
                def _():
                    gather_wait(jl)
                    pltpu.async_copy(
                        rows_v[jl],
                        out_hbm.at[pl.ds(base + (k - _GLAG) * _CHUNK,
                                         _CHUNK)],
                        sem_o[jl])

                @pl.when(k + _GLAG < _NCHUNK)
                def _():
                    idx_start(k + _GLAG, jl)
            return carry

        lax.fori_loop(0, _NITER, body, 0)

        for j in range(_GLAG, _NBUF):
            gather_wait(j)
            out_start(_NCHUNK - _NBUF + j, j)
        for j in range(_NBUF):
            out_wait(j)

    return gather_kernel


_gather = _make_gather()


@jax.jit
def kernel(bank, weight_alloc):
    wa = weight_alloc.reshape(-1).astype(jnp.int32)
    out = _gather(bank, wa)
    return out.reshape(_FULL_ROWS, _FULL_COLS)


# speedup vs baseline: 1.0005x; 1.0005x over previous
import functools

import jax
import jax.numpy as jnp
from jax import lax
from jax.experimental import pallas as pl
from jax.experimental.pallas import tpu as pltpu
from jax.experimental.pallas import tpu_sc as plsc

_FULL_ROWS = 100_000
_FULL_COLS = 128
_N_ALLOC = _FULL_ROWS * _FULL_COLS
_NUM_BANK = 1_280_000
_NUM_CORES = 2
_NUM_SUBCORES = 16
_NW = _NUM_CORES * _NUM_SUBCORES
_PER_W = _N_ALLOC // _NW
_CHUNK = 4_000
_NCHUNK = _PER_W // _CHUNK
_NBUF = 4
_GLAG = _NBUF // 2
_NITER = _NCHUNK // _NBUF
_BANK_SLICE = _NUM_BANK // _NUM_SUBCORES


def _make_gather():
    mesh = plsc.VectorSubcoreMesh(
        core_axis_name="c", subcore_axis_name="s")

    @functools.partial(
        pl.kernel,
        mesh=mesh,
        out_type=jax.ShapeDtypeStruct((_N_ALLOC,), jnp.float32),
        scratch_types=(
            [pltpu.VMEM_SHARED((_NUM_BANK,), jnp.float32)]
            + [pltpu.VMEM((_CHUNK,), jnp.int32) for _ in range(_NBUF)]
            + [pltpu.VMEM((_CHUNK,), jnp.float32) for _ in range(_NBUF)]
            + [pltpu.SemaphoreType.DMA for _ in range(3 * _NBUF)]
        ),
    )
    def gather_kernel(bank_hbm, wa_hbm, out_hbm, bank_sh, *bufs):
        idx_v = bufs[0:_NBUF]
        rows_v = bufs[_NBUF:2 * _NBUF]
        sem_i = bufs[2 * _NBUF:3 * _NBUF]
        sem_o = bufs[3 * _NBUF:4 * _NBUF]
        sem_g = bufs[4 * _NBUF:5 * _NBUF]

        sid = lax.axis_index("s")
        wid = lax.axis_index("c") * _NUM_SUBCORES + sid
        base = wid * _PER_W

        def idx_start(k, j):
            pltpu.async_copy(
                wa_hbm.at[pl.ds(base + k * _CHUNK, _CHUNK)],
                idx_v[j], sem_i[j])

        def idx_wait(j):
            pltpu.make_async_copy(
                wa_hbm.at[pl.ds(base, _CHUNK)], idx_v[j], sem_i[j]).wait()

        def gather_start(j):
            pltpu.async_copy(bank_sh.at[idx_v[j]], rows_v[j], sem_g[j])

        def gather_wait(j):
            pltpu.make_async_copy(
                bank_sh.at[idx_v[j]], rows_v[j], sem_g[j]).wait()

        def out_start(k, j):
            pltpu.async_copy(
                rows_v[j], out_hbm.at[pl.ds(base + k * _CHUNK, _CHUNK)],
                sem_o[j])

        def out_wait(j):
            pltpu.make_async_copy(
                rows_v[j], out_hbm.at[pl.ds(base, _CHUNK)], sem_o[j]).wait()

        for j in range(_GLAG):
            idx_start(j, j)

        boff = sid * _BANK_SLICE
        pltpu.sync_copy(bank_hbm.at[pl.ds(boff, _BANK_SLICE)],
                        bank_sh.at[pl.ds(boff, _BANK_SLICE)])
        plsc.subcore_barrier()

        def body(i, carry):
            for j in range(_NBUF):
                k = i * _NBUF + j
                jl = (j + _GLAG) % _NBUF

                idx_wait(j)

                @pl.when(k >= _NBUF)
                def _():
                    out_wait(j)

                gather_start(j)

                @pl.when(k >= _GLAG)
                def _():
                    gather_wait(jl)
                    pltpu.async_copy(
                        rows_v[jl],
                        out_hbm.at[pl.ds(base + (k - _GLAG) * _CHUNK,
                                         _CHUNK)],
                        sem_o[jl])

                @pl.when(k + _GLAG < _NCHUNK)
                def _():
                    idx_start(k + _GLAG, jl)
            return carry

        lax.fori_loop(0, _NITER, body, 0)

        for j in range(_GLAG, _NBUF):
            gather_wait(j)
            out_start(_NCHUNK - _NBUF + j, j)
        for j in range(_NBUF):
            out_wait(j)

    return gather_kernel


_gather = _make_gather()


@jax.jit
def kernel(bank, weight_alloc):
    wa = weight_alloc.reshape(-1).astype(jnp.int32)
    out = _gather(bank, wa)
    return out.reshape(_FULL_ROWS, _FULL_COLS)
